# baseline (speedup 1.0000x reference)
import functools

import jax
import jax.numpy as jnp
from jax import lax
from jax.experimental import pallas as pl
from jax.experimental.pallas import tpu as pltpu
from jax.experimental.pallas import tpu_sc as plsc

N = 100000
NBX = 256
NBY = 256
BSX = 1.0 / NBX
BSY = 1.0 / NBY
STRETCH = 1.4142135
MINSX = BSX * STRETCH
MINSY = BSY * STRETCH
INV_CAP = 1.0 / 100.0

NW = 32
PER_W = 3136
NPAD = NW * PER_W
NBATCH = PER_W // 16
ROWS = 264
COLS = 272


def _build_sc_scatter():
    mesh = plsc.VectorSubcoreMesh(core_axis_name="c", subcore_axis_name="s")

    @functools.partial(
        pl.kernel,
        mesh=mesh,
        compiler_params=pltpu.CompilerParams(needs_layout_passes=False),
        out_type=jax.ShapeDtypeStruct((NW, ROWS, COLS), jnp.float32),
        scratch_types=[
            pltpu.VMEM((5, PER_W), jnp.float32),
            pltpu.VMEM((ROWS, COLS), jnp.float32),
            pltpu.VMEM((4, 16), jnp.float32),
            pltpu.VMEM((4, 16), jnp.float32),
            pltpu.VMEM((4, 16), jnp.int32),
            pltpu.VMEM((4, 16), jnp.int32),
        ],
    )
    def sc_fn(packed, outp, inbuf, acc, dxs, dys, xps, yps):
        wid = lax.axis_index("s") * 2 + lax.axis_index("c")
        pltpu.sync_copy(packed.at[wid], inbuf)

        zero16 = jnp.zeros((16,), jnp.float32)

        def zrow(r, carry):
            for c in range(COLS // 16):
                acc[r, pl.ds(c * 16, 16)] = zero16
            return carry

        lax.fori_loop(0, ROWS, zrow, 0, unroll=False)

        aidx = jnp.arange(16, dtype=jnp.int32) >> 2
        bidx = jnp.arange(16, dtype=jnp.int32) & 3

        def batch(t, carry):
            o = t * 16
            x = inbuf[0, pl.ds(o, 16)]
            y = inbuf[1, pl.ds(o, 16)]
            w = inbuf[2, pl.ds(o, 16)]
            h = inbuf[3, pl.ds(o, 16)]
            wt = inbuf[4, pl.ds(o, 16)]
            sx = jnp.maximum(w, MINSX)
            sy = jnp.maximum(h, MINSY)
            dens = wt * INV_CAP / (sx * sy)
            ux = x * 256.0 - sx * 128.0
            vx = ux + sx * 256.0
            i0x = (ux + 8.0).astype(jnp.int32) - 8
            fx = ux - i0x.astype(jnp.float32)
            i1x = (vx + 8.0).astype(jnp.int32) - 8
            gx = vx - i1x.astype(jnp.float32)
            cx = (i1x - i0x) == 1
            xps[0, :] = i0x + 4
            xps[1, :] = i0x + 5
            xps[2, :] = jnp.where(cx, i0x + 3, i1x + 4)
            xps[3, :] = i1x + 5
            dxs[0, :] = (1.0 - fx) * dens
            dxs[1, :] = jnp.where(cx, fx - 1.0 + gx, fx) * dens
            dxs[2, :] = jnp.where(cx, 0.0, gx - 1.0) * dens
            dxs[3, :] = -gx * dens
            uy = y * 256.0 - sy * 128.0
            vy = uy + sy * 256.0
            i0y = (uy + 8.0).astype(jnp.int32) - 8
            fy = uy - i0y.astype(jnp.float32)
            i1y = (vy + 8.0).astype(jnp.int32) - 8
            gy = vy - i1y.astype(jnp.float32)
            cy = (i1y - i0y) == 1
            yps[0, :] = i0y + 4
            yps[1, :] = i0y + 5
            yps[2, :] = jnp.where(cy, i0y + 3, i1y + 4)
            yps[3, :] = i1y + 5
            dys[0, :] = 1.0 - fy
            dys[1, :] = jnp.where(cy, fy - 1.0 + gy, fy)
            dys[2, :] = jnp.where(cy, 0.0, gy - 1.0)
            dys[3, :] = -gy
            for j in range(16):
                jv = jnp.full((16,), j, jnp.int32)
                dxv = plsc.load_gather(dxs, [aidx, jv])
                xv = plsc.load_gather(xps, [aidx, jv])
                dyv = plsc.load_gather(dys, [bidx, jv])
                yv = plsc.load_gather(yps, [bidx, jv])
                plsc.addupdate_scatter(acc, [xv, yv], dxv * dyv)
            return carry

        lax.fori_loop(0, NBATCH, batch, 0, unroll=False)
        pltpu.sync_copy(acc, outp.at[wid])

    return sc_fn


def _tc_reduce_body(parts_ref, out_ref):
    s = jnp.sum(parts_ref[...], axis=0)
    c_in = lax.broadcasted_iota(jnp.int32, (NBX, ROWS), 1)
    c_out = lax.broadcasted_iota(jnp.int32, (NBX, ROWS), 0)
    amat = (c_in <= c_out + 4).astype(jnp.float32)
    d_in = lax.broadcasted_iota(jnp.int32, (COLS, NBY), 0)
    d_out = lax.broadcasted_iota(jnp.int32, (COLS, NBY), 1)
    bmat = (d_in <= d_out + 4).astype(jnp.float32)
    t = jax.lax.dot(s, bmat, precision=jax.lax.Precision.HIGHEST)
    out_ref[...] = jax.lax.dot(amat, t, precision=jax.lax.Precision.HIGHEST)


_tc_reduce = pl.pallas_call(
    _tc_reduce_body,
    out_shape=jax.ShapeDtypeStruct((NBX, NBY), jnp.float32),
)


def kernel(inst_sizes, inst_pos, inst_pin_weights):
    pad = NPAD - N
    x = jnp.concatenate([inst_pos[:, 0], jnp.full((pad,), 0.5, jnp.float32)])
    y = jnp.concatenate([inst_pos[:, 1], jnp.full((pad,), 0.5, jnp.float32)])
    w = jnp.concatenate([inst_sizes[:, 0], jnp.full((pad,), 0.5, jnp.float32)])
    h = jnp.concatenate([inst_sizes[:, 1], jnp.full((pad,), 0.5, jnp.float32)])
    wt = jnp.concatenate([inst_pin_weights, jnp.zeros((pad,), jnp.float32)])
    packed = jnp.stack([x, y, w, h, wt], axis=0)
    packed = packed.reshape(5, NW, PER_W).transpose(1, 0, 2)
    parts = _build_sc_scatter()(packed)
    return _tc_reduce(parts)


# speedup vs baseline: 2.1359x; 2.1359x over previous
import functools

import jax
import jax.numpy as jnp
from jax import lax
from jax.experimental import pallas as pl
from jax.experimental.pallas import tpu as pltpu
from jax.experimental.pallas import tpu_sc as plsc

N = 100000
NBX = 256
NBY = 256
BSX = 1.0 / NBX
BSY = 1.0 / NBY
STRETCH = 1.4142135
MINSX = BSX * STRETCH
MINSY = BSY * STRETCH
INV_CAP = 1.0 / 100.0

NW = 32
PER_W = 3136
NPAD = NW * PER_W
NBATCH = PER_W // 16
ROWS = 264
SROW = 273
ACCW = ROWS * SROW
ACCPAD = 73728
FLATW = 5 * PER_W


def _build_sc_scatter():
    mesh = plsc.VectorSubcoreMesh(core_axis_name="c", subcore_axis_name="s")

    @functools.partial(
        pl.kernel,
        mesh=mesh,
        compiler_params=pltpu.CompilerParams(needs_layout_passes=False),
        out_type=jax.ShapeDtypeStruct((NW, ACCPAD), jnp.float32),
        scratch_types=[
            pltpu.VMEM((FLATW,), jnp.float32),
            pltpu.VMEM((ACCPAD,), jnp.float32),
            pltpu.VMEM((272,), jnp.float32),
            pltpu.VMEM((272,), jnp.int32),
            pltpu.SemaphoreType.DMA,
        ],
    )
    def sc_fn(packed, outp, inbuf, acc, vbuf, ibuf, sem):
        wid = lax.axis_index("s") * 2 + lax.axis_index("c")
        cp = pltpu.async_copy(packed.at[wid], inbuf, sem)

        zero16 = jnp.zeros((16,), jnp.float32)

        def zblock(r, carry):
            for k in range(16):
                acc[pl.ds(r * 256 + k * 16, 16)] = zero16
            return carry

        lax.fori_loop(0, ACCPAD // 256, zblock, 0, unroll=False)
        cp.wait()

        iota = jnp.arange(16, dtype=jnp.int32)
        tidx = [iota * 17 + q for q in range(16)]

        def batch(t, carry):
            o = t * 16
            x = inbuf[pl.ds(o, 16)]
            y = inbuf[pl.ds(PER_W + o, 16)]
            w = inbuf[pl.ds(2 * PER_W + o, 16)]
            h = inbuf[pl.ds(3 * PER_W + o, 16)]
            wt = inbuf[pl.ds(4 * PER_W + o, 16)]
            sx = jnp.maximum(w, MINSX)
            sy = jnp.maximum(h, MINSY)
            dens = wt * INV_CAP / (sx * sy)
            ux = x * 256.0 - sx * 128.0
            vx = ux + sx * 256.0
            i0x = (ux + 8.0).astype(jnp.int32) - 8
            fx = ux - i0x.astype(jnp.float32)
            i1x = (vx + 8.0).astype(jnp.int32) - 8
            gx = vx - i1x.astype(jnp.float32)
            cx = (i1x - i0x) == 1
            xm = [
                (i0x + 4) * SROW,
                (i0x + 5) * SROW,
                jnp.where(cx, i0x + 3, i1x + 4) * SROW,
                (i1x + 5) * SROW,
            ]
            dxd = [
                (1.0 - fx) * dens,
                jnp.where(cx, fx - 1.0 + gx, fx) * dens,
                jnp.where(cx, 0.0, gx - 1.0) * dens,
                -gx * dens,
            ]
            uy = y * 256.0 - sy * 128.0
            vy = uy + sy * 256.0
            i0y = (uy + 8.0).astype(jnp.int32) - 8
            fy = uy - i0y.astype(jnp.float32)
            i1y = (vy + 8.0).astype(jnp.int32) - 8
            gy = vy - i1y.astype(jnp.float32)
            cy = (i1y - i0y) == 1
            yp = [
                i0y + 4,
                i0y + 5,
                jnp.where(cy, i0y + 3, i1y + 4),
                i1y + 5,
            ]
            dy = [
                1.0 - fy,
                jnp.where(cy, fy - 1.0 + gy, fy),
                jnp.where(cy, 0.0, gy - 1.0),
                -gy,
            ]
            for q in range(16):
                a, b = q >> 2, q & 3
                plsc.store_scatter(ibuf, [tidx[q]], xm[a] + yp[b])
                plsc.store_scatter(vbuf, [tidx[q]], dxd[a] * dy[b])
            for j in range(16):
                iv = ibuf[pl.ds(17 * j, 16)]
                vv = vbuf[pl.ds(17 * j, 16)]
                plsc.addupdate_scatter(acc, [iv], vv)
            return carry

        lax.fori_loop(0, NBATCH, batch, 0, unroll=False)
        pltpu.sync_copy(acc, outp.at[wid])

    return sc_fn


def _tc_reduce_body(parts_ref, out_ref):
    s = jnp.sum(parts_ref[...], axis=0)
    c_in = lax.broadcasted_iota(jnp.int32, (NBX, ROWS), 1)
    c_out = lax.broadcasted_iota(jnp.int32, (NBX, ROWS), 0)
    amat = (c_in <= c_out + 4).astype(jnp.float32)
    d_in = lax.broadcasted_iota(jnp.int32, (SROW, NBY), 0)
    d_out = lax.broadcasted_iota(jnp.int32, (SROW, NBY), 1)
    bmat = (d_in <= d_out + 4).astype(jnp.float32)
    t = jax.lax.dot(s, bmat, precision=jax.lax.Precision.HIGHEST)
    out_ref[...] = jax.lax.dot(amat, t, precision=jax.lax.Precision.HIGHEST)


_tc_reduce = pl.pallas_call(
    _tc_reduce_body,
    out_shape=jax.ShapeDtypeStruct((NBX, NBY), jnp.float32),
)


def kernel(inst_sizes, inst_pos, inst_pin_weights):
    pad = NPAD - N
    x = jnp.concatenate([inst_pos[:, 0], jnp.full((pad,), 0.5, jnp.float32)])
    y = jnp.concatenate([inst_pos[:, 1], jnp.full((pad,), 0.5, jnp.float32)])
    w = jnp.concatenate([inst_sizes[:, 0], jnp.full((pad,), 0.5, jnp.float32)])
    h = jnp.concatenate([inst_sizes[:, 1], jnp.full((pad,), 0.5, jnp.float32)])
    wt = jnp.concatenate([inst_pin_weights, jnp.zeros((pad,), jnp.float32)])
    packed = jnp.stack([x, y, w, h, wt], axis=0)
    packed = packed.reshape(5, NW, PER_W).transpose(1, 0, 2)
    packed = packed.reshape(NW, FLATW)
    parts = _build_sc_scatter()(packed)
    parts = parts[:, :ACCW].reshape(NW, ROWS, SROW)
    return _tc_reduce(parts)
